# Initial kernel scaffold; baseline (speedup 1.0000x reference)
import jax
import jax.numpy as jnp
from jax.experimental import pallas as pl


def kernel(x, edge_index, W1, b1, W2, b2):
    raise NotImplementedError("write your pallas kernel here")


# speedup vs baseline: 46.2349x; 46.2349x over previous
import functools

import jax
import jax.numpy as jnp
from jax import lax
from jax.experimental import pallas as pl
from jax.experimental.pallas import tpu as pltpu
from jax.experimental.pallas import tpu_sc as plsc

N = 100000
E = 6400000
N_PAD = 100352
ROWS_PER_TILE = N_PAD // 16

NW = 32
B = 100
CH = 16
EROWS = E // B
ROWS_PER_W = EROWS // NW
N_SUPER = ROWS_PER_W // CH

@functools.lru_cache(maxsize=None)
def _sc_mesh():
    return plsc.VectorSubcoreMesh(core_axis_name="c", subcore_axis_name="s")


def _deg_body(eidx, ones, zeros, out, dst_v, ones_v, acc_sh, gsem):
    cid = lax.axis_index("c")
    sid = lax.axis_index("s")
    w = cid * 16 + sid
    sl = pl.ds(sid * ROWS_PER_TILE, ROWS_PER_TILE)
    pltpu.sync_copy(zeros, acc_sh.at[sl])
    pltpu.sync_copy(ones, ones_v)
    plsc.subcore_barrier()

    def body(i, _):
        row = w * ROWS_PER_W + i * CH
        pltpu.sync_copy(eidx.at[1, pl.ds(row, CH)], dst_v)
        for j in range(CH):
            pltpu.sync_copy(ones_v, acc_sh.at[dst_v.at[j]], add=True)
        return 0

    lax.fori_loop(0, N_SUPER, body, 0)
    plsc.subcore_barrier()
    pltpu.sync_copy(acc_sh.at[sl], out.at[cid, sl])


@functools.lru_cache(maxsize=None)
def _deg_call():
    return pl.kernel(
        _deg_body,
        out_type=jax.ShapeDtypeStruct((2, N_PAD), jnp.float32),
        mesh=_sc_mesh(),
        compiler_params=pltpu.CompilerParams(use_tc_tiling_on_sc=False),
        scratch_types=[
            pltpu.VMEM((CH, B), jnp.int32),
            pltpu.VMEM((B,), jnp.float32),
            pltpu.VMEM_SHARED((N_PAD,), jnp.float32),
            pltpu.SemaphoreType.DMA,
        ],
    )


def _make_scatter(F):
    def body_fn(eidx, p, zeros, out, src_v, dst_v, rows_v, acc_sh, gsem):
        cid = lax.axis_index("c")
        sid = lax.axis_index("s")
        w = cid * 16 + sid
        sl = pl.ds(sid * ROWS_PER_TILE, ROWS_PER_TILE)
        pltpu.sync_copy(zeros, acc_sh.at[sl])
        plsc.subcore_barrier()

        def body(i, _):
            row = w * ROWS_PER_W + i * CH
            pltpu.sync_copy(eidx.at[0, pl.ds(row, CH)], src_v)
            pltpu.sync_copy(eidx.at[1, pl.ds(row, CH)], dst_v)
            cps = [
                pltpu.async_copy(p.at[src_v.at[j]], rows_v.at[j], gsem)
                for j in range(CH)
            ]
            for c in cps:
                c.wait()
            for j in range(CH):
                pltpu.sync_copy(rows_v.at[j], acc_sh.at[dst_v.at[j]], add=True)
            return 0

        lax.fori_loop(0, N_SUPER, body, 0)
        plsc.subcore_barrier()
        pltpu.sync_copy(acc_sh.at[sl], out.at[cid, sl])

    return pl.kernel(
        body_fn,
        out_type=jax.ShapeDtypeStruct((2, N_PAD, F), jnp.float32),
        mesh=_sc_mesh(),
        compiler_params=pltpu.CompilerParams(use_tc_tiling_on_sc=False),
        scratch_types=[
            pltpu.VMEM((CH, B), jnp.int32),
            pltpu.VMEM((CH, B), jnp.int32),
            pltpu.VMEM((CH, B, F), jnp.float32),
            pltpu.VMEM_SHARED((N_PAD, F), jnp.float32),
            pltpu.SemaphoreType.DMA,
        ],
    )


_scatter_call = functools.lru_cache(maxsize=None)(_make_scatter)


R = 1024
G = N_PAD // R


def _matmul_small(a, w):
    acc = a[:, 0:1] * w[0:1, :]
    for k in range(1, w.shape[0]):
        acc = acc + a[:, k : k + 1] * w[k : k + 1, :]
    return acc


def _tc1_body(degs_ref, x_ref, w1_ref, dinv_ref, h1_ref, p1_ref):
    deg = degs_ref[0] + degs_ref[1] + 1.0
    dinv = lax.rsqrt(deg)
    h1 = _matmul_small(x_ref[...], w1_ref[...])
    dinv_ref[...] = dinv
    h1_ref[...] = h1
    p1_ref[...] = dinv * h1


def _tc2_body(acc_ref, dinv_ref, h1_ref, w2_ref, b1_ref, h2_ref, p2_ref):
    dinv = dinv_ref[...]
    accs = acc_ref[0] + acc_ref[1]
    z1 = dinv * accs + dinv * dinv * h1_ref[...] + b1_ref[...]
    a = jnp.maximum(z1, 0.0)
    h2 = _matmul_small(a, w2_ref[...])
    h2_ref[...] = h2
    p2 = dinv * h2
    p2_ref[...] = jnp.concatenate([p2, jnp.zeros((R, 6), jnp.float32)], axis=1)


def _tc3_body(acc_ref, dinv_ref, h2_ref, b2_ref, out_ref):
    dinv = dinv_ref[...]
    accs = acc_ref[0, :, :2] + acc_ref[1, :, :2]
    z = dinv * accs + dinv * dinv * h2_ref[...] + b2_ref[...]
    m = jnp.max(z, axis=1, keepdims=True)
    out_ref[...] = z - m - jnp.log(jnp.sum(jnp.exp(z - m), axis=1, keepdims=True))


def _row_spec(f):
    return pl.BlockSpec((R, f), lambda i: (i, 0))


def _pair_spec(f):
    return pl.BlockSpec((2, R, f), lambda i: (0, i, 0))


def _const_spec(shape):
    return pl.BlockSpec(shape, lambda i: tuple(0 for _ in shape))


def _tc1(degs, x, w1):
    return pl.pallas_call(
        _tc1_body,
        grid=(G,),
        in_specs=[_pair_spec(1), _row_spec(3), _const_spec((3, 8))],
        out_specs=[_row_spec(1), _row_spec(8), _row_spec(8)],
        out_shape=[
            jax.ShapeDtypeStruct((N_PAD, 1), jnp.float32),
            jax.ShapeDtypeStruct((N_PAD, 8), jnp.float32),
            jax.ShapeDtypeStruct((N_PAD, 8), jnp.float32),
        ],
    )(degs, x, w1)


def _tc2(acc, dinv, h1, w2, b1):
    return pl.pallas_call(
        _tc2_body,
        grid=(G,),
        in_specs=[
            _pair_spec(8), _row_spec(1), _row_spec(8),
            _const_spec((8, 2)), _const_spec((1, 8)),
        ],
        out_specs=[_row_spec(2), _row_spec(8)],
        out_shape=[
            jax.ShapeDtypeStruct((N_PAD, 2), jnp.float32),
            jax.ShapeDtypeStruct((N_PAD, 8), jnp.float32),
        ],
    )(acc, dinv, h1, w2, b1)


def _tc3(acc, dinv, h2, b2):
    return pl.pallas_call(
        _tc3_body,
        grid=(G,),
        in_specs=[
            _pair_spec(8), _row_spec(1), _row_spec(2), _const_spec((1, 2)),
        ],
        out_specs=_row_spec(2),
        out_shape=jax.ShapeDtypeStruct((N_PAD, 2), jnp.float32),
    )(acc, dinv, h2, b2)


@jax.jit
def kernel(x, edge_index, W1, b1, W2, b2):
    ei = edge_index.astype(jnp.int32).reshape(2, EROWS, B)
    x_p = jnp.pad(x, ((0, N_PAD - N), (0, 0)))
    zeros1 = jnp.zeros((ROWS_PER_TILE,), jnp.float32)
    zeros8 = jnp.zeros((ROWS_PER_TILE, 8), jnp.float32)
    ones = jnp.ones((B,), jnp.float32)

    degs = _deg_call()(ei, ones, zeros1)
    degs = degs.reshape(2, N_PAD, 1)
    dinv, h1, p1 = _tc1(degs, x_p, W1)
    acc1 = _scatter_call(8)(ei, p1, zeros8)
    h2, p2 = _tc2(acc1, dinv, h1, W2, b1.reshape(1, 8))
    acc2 = _scatter_call(8)(ei, p2, zeros8)
    out = _tc3(acc2, dinv, h2, b2.reshape(1, 2))
    return out[:N]


# speedup vs baseline: 53.3564x; 1.1540x over previous
import functools

import jax
import jax.numpy as jnp
from jax import lax
from jax.experimental import pallas as pl
from jax.experimental.pallas import tpu as pltpu
from jax.experimental.pallas import tpu_sc as plsc

N = 100000
E = 6400000
N_PAD = 100352
ROWS_PER_TILE = N_PAD // 16

NW = 32
B = 100
CH = 16
EROWS = E // B
ROWS_PER_W = EROWS // NW
N_SUPER = ROWS_PER_W // CH

@functools.lru_cache(maxsize=None)
def _sc_mesh():
    return plsc.VectorSubcoreMesh(core_axis_name="c", subcore_axis_name="s")


def _deg_body(eidx, ones, zeros, out, dst_v, ones_v, acc_sh, ssem):
    cid = lax.axis_index("c")
    sid = lax.axis_index("s")
    w = cid * 16 + sid
    sl = pl.ds(sid * ROWS_PER_TILE, ROWS_PER_TILE)
    pltpu.sync_copy(zeros, acc_sh.at[sl])
    pltpu.sync_copy(ones, ones_v)
    plsc.subcore_barrier()

    def body(i, _):
        row = w * ROWS_PER_W + i * CH
        pltpu.sync_copy(eidx.at[1, pl.ds(row, CH)], dst_v)
        sps = [
            pltpu.async_copy(ones_v, acc_sh.at[dst_v.at[j]], ssem, add=True)
            for j in range(CH)
        ]
        for s in sps:
            s.wait()
        return 0

    lax.fori_loop(0, N_SUPER, body, 0)
    plsc.subcore_barrier()
    pltpu.sync_copy(acc_sh.at[sl], out.at[cid, sl])


@functools.lru_cache(maxsize=None)
def _deg_call():
    return pl.kernel(
        _deg_body,
        out_type=jax.ShapeDtypeStruct((2, N_PAD), jnp.float32),
        mesh=_sc_mesh(),
        compiler_params=pltpu.CompilerParams(use_tc_tiling_on_sc=False),
        scratch_types=[
            pltpu.VMEM((CH, B), jnp.int32),
            pltpu.VMEM((B,), jnp.float32),
            pltpu.VMEM_SHARED((N_PAD,), jnp.float32),
            pltpu.SemaphoreType.DMA,
        ],
    )


def _make_scatter(F):
    def body_fn(eidx, p, zeros, out, src_v, dst_v, rows_v, acc_sh, gsem, ssem):
        cid = lax.axis_index("c")
        sid = lax.axis_index("s")
        w = cid * 16 + sid
        sl = pl.ds(sid * ROWS_PER_TILE, ROWS_PER_TILE)
        pltpu.sync_copy(zeros, acc_sh.at[sl])
        plsc.subcore_barrier()

        def body(i, _):
            row = w * ROWS_PER_W + i * CH
            i0 = pltpu.async_copy(eidx.at[0, pl.ds(row, CH)], src_v, gsem)
            i1 = pltpu.async_copy(eidx.at[1, pl.ds(row, CH)], dst_v, gsem)
            i0.wait()
            i1.wait()
            cps = [
                pltpu.async_copy(p.at[src_v.at[j]], rows_v.at[j], gsem)
                for j in range(CH)
            ]
            for c in cps:
                c.wait()
            sps = [
                pltpu.async_copy(rows_v.at[j], acc_sh.at[dst_v.at[j]], ssem, add=True)
                for j in range(CH)
            ]
            for s in sps:
                s.wait()
            return 0

        lax.fori_loop(0, N_SUPER, body, 0)
        plsc.subcore_barrier()
        pltpu.sync_copy(acc_sh.at[sl], out.at[cid, sl])

    return pl.kernel(
        body_fn,
        out_type=jax.ShapeDtypeStruct((2, N_PAD, F), jnp.float32),
        mesh=_sc_mesh(),
        compiler_params=pltpu.CompilerParams(use_tc_tiling_on_sc=False),
        scratch_types=[
            pltpu.VMEM((CH, B), jnp.int32),
            pltpu.VMEM((CH, B), jnp.int32),
            pltpu.VMEM((CH, B, F), jnp.float32),
            pltpu.VMEM_SHARED((N_PAD, F), jnp.float32),
            pltpu.SemaphoreType.DMA,
            pltpu.SemaphoreType.DMA,
        ],
    )


_scatter_call = functools.lru_cache(maxsize=None)(_make_scatter)


R = 1024
G = N_PAD // R


def _matmul_small(a, w):
    acc = a[:, 0:1] * w[0:1, :]
    for k in range(1, w.shape[0]):
        acc = acc + a[:, k : k + 1] * w[k : k + 1, :]
    return acc


def _tc1_body(degs_ref, x_ref, w1_ref, dinv_ref, h1_ref, p1_ref):
    deg = degs_ref[0] + degs_ref[1] + 1.0
    dinv = lax.rsqrt(deg)
    h1 = _matmul_small(x_ref[...], w1_ref[...])
    dinv_ref[...] = dinv
    h1_ref[...] = h1
    p1_ref[...] = dinv * h1


def _tc2_body(acc_ref, dinv_ref, h1_ref, w2_ref, b1_ref, h2_ref, p2_ref):
    dinv = dinv_ref[...]
    accs = acc_ref[0] + acc_ref[1]
    z1 = dinv * accs + dinv * dinv * h1_ref[...] + b1_ref[...]
    a = jnp.maximum(z1, 0.0)
    h2 = _matmul_small(a, w2_ref[...])
    h2_ref[...] = h2
    p2 = dinv * h2
    p2_ref[...] = jnp.concatenate([p2, jnp.zeros((R, 6), jnp.float32)], axis=1)


def _tc3_body(acc_ref, dinv_ref, h2_ref, b2_ref, out_ref):
    dinv = dinv_ref[...]
    accs = acc_ref[0, :, :2] + acc_ref[1, :, :2]
    z = dinv * accs + dinv * dinv * h2_ref[...] + b2_ref[...]
    m = jnp.max(z, axis=1, keepdims=True)
    out_ref[...] = z - m - jnp.log(jnp.sum(jnp.exp(z - m), axis=1, keepdims=True))


def _row_spec(f):
    return pl.BlockSpec((R, f), lambda i: (i, 0))


def _pair_spec(f):
    return pl.BlockSpec((2, R, f), lambda i: (0, i, 0))


def _const_spec(shape):
    return pl.BlockSpec(shape, lambda i: tuple(0 for _ in shape))


def _tc1(degs, x, w1):
    return pl.pallas_call(
        _tc1_body,
        grid=(G,),
        in_specs=[_pair_spec(1), _row_spec(3), _const_spec((3, 8))],
        out_specs=[_row_spec(1), _row_spec(8), _row_spec(8)],
        out_shape=[
            jax.ShapeDtypeStruct((N_PAD, 1), jnp.float32),
            jax.ShapeDtypeStruct((N_PAD, 8), jnp.float32),
            jax.ShapeDtypeStruct((N_PAD, 8), jnp.float32),
        ],
    )(degs, x, w1)


def _tc2(acc, dinv, h1, w2, b1):
    return pl.pallas_call(
        _tc2_body,
        grid=(G,),
        in_specs=[
            _pair_spec(8), _row_spec(1), _row_spec(8),
            _const_spec((8, 2)), _const_spec((1, 8)),
        ],
        out_specs=[_row_spec(2), _row_spec(8)],
        out_shape=[
            jax.ShapeDtypeStruct((N_PAD, 2), jnp.float32),
            jax.ShapeDtypeStruct((N_PAD, 8), jnp.float32),
        ],
    )(acc, dinv, h1, w2, b1)


def _tc3(acc, dinv, h2, b2):
    return pl.pallas_call(
        _tc3_body,
        grid=(G,),
        in_specs=[
            _pair_spec(8), _row_spec(1), _row_spec(2), _const_spec((1, 2)),
        ],
        out_specs=_row_spec(2),
        out_shape=jax.ShapeDtypeStruct((N_PAD, 2), jnp.float32),
    )(acc, dinv, h2, b2)


@jax.jit
def kernel(x, edge_index, W1, b1, W2, b2):
    ei = edge_index.astype(jnp.int32).reshape(2, EROWS, B)
    x_p = jnp.pad(x, ((0, N_PAD - N), (0, 0)))
    zeros1 = jnp.zeros((ROWS_PER_TILE,), jnp.float32)
    zeros8 = jnp.zeros((ROWS_PER_TILE, 8), jnp.float32)
    ones = jnp.ones((B,), jnp.float32)

    degs = _deg_call()(ei, ones, zeros1)
    degs = degs.reshape(2, N_PAD, 1)
    dinv, h1, p1 = _tc1(degs, x_p, W1)
    acc1 = _scatter_call(8)(ei, p1, zeros8)
    h2, p2 = _tc2(acc1, dinv, h1, W2, b1.reshape(1, 8))
    acc2 = _scatter_call(8)(ei, p2, zeros8)
    out = _tc3(acc2, dinv, h2, b2.reshape(1, 2))
    return out[:N]


# speedup vs baseline: 61.6849x; 1.1561x over previous
import functools

import jax
import jax.numpy as jnp
from jax import lax
from jax.experimental import pallas as pl
from jax.experimental.pallas import tpu as pltpu
from jax.experimental.pallas import tpu_sc as plsc

N = 100000
E = 6400000
N_PAD = 100352
ROWS_PER_TILE = N_PAD // 16

NW = 32
B = 100
CH = 16
EROWS = E // B
ROWS_PER_W = EROWS // NW
N_SUPER = ROWS_PER_W // CH

@functools.lru_cache(maxsize=None)
def _sc_mesh():
    return plsc.VectorSubcoreMesh(core_axis_name="c", subcore_axis_name="s")


def _deg_body(eidx, ones, zeros, out, dst_v, ones_v, acc_sh, ssem):
    cid = lax.axis_index("c")
    sid = lax.axis_index("s")
    w = cid * 16 + sid
    sl = pl.ds(sid * ROWS_PER_TILE, ROWS_PER_TILE)
    pltpu.sync_copy(zeros, acc_sh.at[sl])
    pltpu.sync_copy(ones, ones_v)
    plsc.subcore_barrier()

    def body(i, _):
        row = w * ROWS_PER_W + i * CH
        pltpu.sync_copy(eidx.at[1, pl.ds(row, CH)], dst_v)
        sps = [
            pltpu.async_copy(ones_v, acc_sh.at[dst_v.at[j]], ssem, add=True)
            for j in range(CH)
        ]
        for s in sps:
            s.wait()
        return 0

    lax.fori_loop(0, N_SUPER, body, 0)
    plsc.subcore_barrier()
    pltpu.sync_copy(acc_sh.at[sl], out.at[cid, sl])


@functools.lru_cache(maxsize=None)
def _deg_call():
    return pl.kernel(
        _deg_body,
        out_type=jax.ShapeDtypeStruct((2, N_PAD), jnp.float32),
        mesh=_sc_mesh(),
        compiler_params=pltpu.CompilerParams(use_tc_tiling_on_sc=False),
        scratch_types=[
            pltpu.VMEM((CH, B), jnp.int32),
            pltpu.VMEM((B,), jnp.float32),
            pltpu.VMEM_SHARED((N_PAD,), jnp.float32),
            pltpu.SemaphoreType.DMA,
        ],
    )


def _make_scatter(F):
    def body_fn(eidx, p, zeros, out, src_v, dst_v, rows_v, tab_sh, acc_sh, gsem, ssem):
        cid = lax.axis_index("c")
        sid = lax.axis_index("s")
        w = cid * 16 + sid
        sl = pl.ds(sid * ROWS_PER_TILE, ROWS_PER_TILE)
        pltpu.sync_copy(zeros, acc_sh.at[sl])
        pltpu.sync_copy(p.at[sl], tab_sh.at[sl])
        plsc.subcore_barrier()

        def body(i, _):
            row = w * ROWS_PER_W + i * CH
            i0 = pltpu.async_copy(eidx.at[0, pl.ds(row, CH)], src_v, gsem)
            i1 = pltpu.async_copy(eidx.at[1, pl.ds(row, CH)], dst_v, gsem)
            i0.wait()
            i1.wait()
            cps = [
                pltpu.async_copy(tab_sh.at[src_v.at[j]], rows_v.at[j], gsem)
                for j in range(CH)
            ]
            for c in cps:
                c.wait()
            sps = [
                pltpu.async_copy(rows_v.at[j], acc_sh.at[dst_v.at[j]], ssem, add=True)
                for j in range(CH)
            ]
            for s in sps:
                s.wait()
            return 0

        lax.fori_loop(0, N_SUPER, body, 0)
        plsc.subcore_barrier()
        pltpu.sync_copy(acc_sh.at[sl], out.at[cid, sl])

    return pl.kernel(
        body_fn,
        out_type=jax.ShapeDtypeStruct((2, N_PAD, F), jnp.float32),
        mesh=_sc_mesh(),
        compiler_params=pltpu.CompilerParams(use_tc_tiling_on_sc=False),
        scratch_types=[
            pltpu.VMEM((CH, B), jnp.int32),
            pltpu.VMEM((CH, B), jnp.int32),
            pltpu.VMEM((CH, B, F), jnp.float32),
            pltpu.VMEM_SHARED((N_PAD, F), jnp.float32),
            pltpu.VMEM_SHARED((N_PAD, F), jnp.float32),
            pltpu.SemaphoreType.DMA,
            pltpu.SemaphoreType.DMA,
        ],
    )


_scatter_call = functools.lru_cache(maxsize=None)(_make_scatter)


R = 1024
G = N_PAD // R


def _matmul_small(a, w):
    acc = a[:, 0:1] * w[0:1, :]
    for k in range(1, w.shape[0]):
        acc = acc + a[:, k : k + 1] * w[k : k + 1, :]
    return acc


def _tc1_body(degs_ref, x_ref, w1_ref, dinv_ref, h1_ref, p1_ref):
    deg = degs_ref[0] + degs_ref[1] + 1.0
    dinv = lax.rsqrt(deg)
    h1 = _matmul_small(x_ref[...], w1_ref[...])
    dinv_ref[...] = dinv
    h1_ref[...] = h1
    p1_ref[...] = dinv * h1


def _tc2_body(acc_ref, dinv_ref, h1_ref, w2_ref, b1_ref, h2_ref, p2_ref):
    dinv = dinv_ref[...]
    accs = acc_ref[0] + acc_ref[1]
    z1 = dinv * accs + dinv * dinv * h1_ref[...] + b1_ref[...]
    a = jnp.maximum(z1, 0.0)
    h2 = _matmul_small(a, w2_ref[...])
    h2_ref[...] = h2
    p2 = dinv * h2
    p2_ref[...] = jnp.concatenate([p2, jnp.zeros((R, 6), jnp.float32)], axis=1)


def _tc3_body(acc_ref, dinv_ref, h2_ref, b2_ref, out_ref):
    dinv = dinv_ref[...]
    accs = acc_ref[0, :, :2] + acc_ref[1, :, :2]
    z = dinv * accs + dinv * dinv * h2_ref[...] + b2_ref[...]
    m = jnp.max(z, axis=1, keepdims=True)
    out_ref[...] = z - m - jnp.log(jnp.sum(jnp.exp(z - m), axis=1, keepdims=True))


def _row_spec(f):
    return pl.BlockSpec((R, f), lambda i: (i, 0))


def _pair_spec(f):
    return pl.BlockSpec((2, R, f), lambda i: (0, i, 0))


def _const_spec(shape):
    return pl.BlockSpec(shape, lambda i: tuple(0 for _ in shape))


def _tc1(degs, x, w1):
    return pl.pallas_call(
        _tc1_body,
        grid=(G,),
        in_specs=[_pair_spec(1), _row_spec(3), _const_spec((3, 8))],
        out_specs=[_row_spec(1), _row_spec(8), _row_spec(8)],
        out_shape=[
            jax.ShapeDtypeStruct((N_PAD, 1), jnp.float32),
            jax.ShapeDtypeStruct((N_PAD, 8), jnp.float32),
            jax.ShapeDtypeStruct((N_PAD, 8), jnp.float32),
        ],
    )(degs, x, w1)


def _tc2(acc, dinv, h1, w2, b1):
    return pl.pallas_call(
        _tc2_body,
        grid=(G,),
        in_specs=[
            _pair_spec(8), _row_spec(1), _row_spec(8),
            _const_spec((8, 2)), _const_spec((1, 8)),
        ],
        out_specs=[_row_spec(2), _row_spec(8)],
        out_shape=[
            jax.ShapeDtypeStruct((N_PAD, 2), jnp.float32),
            jax.ShapeDtypeStruct((N_PAD, 8), jnp.float32),
        ],
    )(acc, dinv, h1, w2, b1)


def _tc3(acc, dinv, h2, b2):
    return pl.pallas_call(
        _tc3_body,
        grid=(G,),
        in_specs=[
            _pair_spec(8), _row_spec(1), _row_spec(2), _const_spec((1, 2)),
        ],
        out_specs=_row_spec(2),
        out_shape=jax.ShapeDtypeStruct((N_PAD, 2), jnp.float32),
    )(acc, dinv, h2, b2)


@jax.jit
def kernel(x, edge_index, W1, b1, W2, b2):
    ei = edge_index.astype(jnp.int32).reshape(2, EROWS, B)
    x_p = jnp.pad(x, ((0, N_PAD - N), (0, 0)))
    zeros1 = jnp.zeros((ROWS_PER_TILE,), jnp.float32)
    zeros8 = jnp.zeros((ROWS_PER_TILE, 8), jnp.float32)
    ones = jnp.ones((B,), jnp.float32)

    degs = _deg_call()(ei, ones, zeros1)
    degs = degs.reshape(2, N_PAD, 1)
    dinv, h1, p1 = _tc1(degs, x_p, W1)
    acc1 = _scatter_call(8)(ei, p1, zeros8)
    h2, p2 = _tc2(acc1, dinv, h1, W2, b1.reshape(1, 8))
    acc2 = _scatter_call(8)(ei, p2, zeros8)
    out = _tc3(acc2, dinv, h2, b2.reshape(1, 2))
    return out[:N]


# speedup vs baseline: 73.7933x; 1.1963x over previous
import functools

import jax
import jax.numpy as jnp
from jax import lax
from jax.experimental import pallas as pl
from jax.experimental.pallas import tpu as pltpu
from jax.experimental.pallas import tpu_sc as plsc

N = 100000
E = 6400000
N_PAD = 100352
ROWS_PER_TILE = N_PAD // 16

NW = 32
B = 100
CH = 16
EROWS = E // B
ROWS_PER_W = EROWS // NW
N_SUPER = ROWS_PER_W // CH

_F = 8


@functools.lru_cache(maxsize=None)
def _sc_mesh():
    return plsc.VectorSubcoreMesh(core_axis_name="c", subcore_axis_name="s")


def _deg_body(eidx, ones, zeros, out, dst_v, ones_v, acc_sh, ssem):
    cid = lax.axis_index("c")
    sid = lax.axis_index("s")
    w = cid * 16 + sid
    sl = pl.ds(sid * ROWS_PER_TILE, ROWS_PER_TILE)
    pltpu.sync_copy(zeros, acc_sh.at[sl])
    pltpu.sync_copy(ones, ones_v)
    plsc.subcore_barrier()

    def body(i, _):
        row = w * ROWS_PER_W + i * CH
        pltpu.sync_copy(eidx.at[1, pl.ds(row, CH)], dst_v)
        sps = [
            pltpu.async_copy(ones_v, acc_sh.at[dst_v.at[j]], ssem, add=True)
            for j in range(CH)
        ]
        for s in sps:
            s.wait()
        return 0

    lax.fori_loop(0, N_SUPER, body, 0)
    plsc.subcore_barrier()
    pltpu.sync_copy(acc_sh.at[sl], out.at[cid, sl])


@functools.lru_cache(maxsize=None)
def _deg_call():
    return pl.kernel(
        _deg_body,
        out_type=jax.ShapeDtypeStruct((2, N_PAD), jnp.float32),
        mesh=_sc_mesh(),
        compiler_params=pltpu.CompilerParams(use_tc_tiling_on_sc=False),
        scratch_types=[
            pltpu.VMEM((CH, B), jnp.int32),
            pltpu.VMEM((B,), jnp.float32),
            pltpu.VMEM_SHARED((N_PAD,), jnp.float32),
            pltpu.SemaphoreType.DMA,
        ],
    )


def _scatter_body(
    eidx, p, zeros, out,
    src_v, dst_v, rows_v, tab_sh, acc_sh, gsem, ssem,
):
    cid = lax.axis_index("c")
    sid = lax.axis_index("s")
    w = cid * 16 + sid
    sl = pl.ds(sid * ROWS_PER_TILE, ROWS_PER_TILE)
    pltpu.sync_copy(zeros, acc_sh.at[sl])
    pltpu.sync_copy(p.at[sl], tab_sh.at[sl])
    plsc.subcore_barrier()

    def body(i, _):
        row = w * ROWS_PER_W + i * CH
        i0 = pltpu.async_copy(eidx.at[0, pl.ds(row, CH)], src_v, gsem)
        i1 = pltpu.async_copy(eidx.at[1, pl.ds(row, CH)], dst_v, gsem)
        i0.wait()
        i1.wait()
        cps = [
            pltpu.async_copy(tab_sh.at[src_v.at[j]], rows_v.at[j], gsem)
            for j in range(CH)
        ]
        for c in cps:
            c.wait()
        sps = [
            pltpu.async_copy(rows_v.at[j], acc_sh.at[dst_v.at[j]], ssem, add=True)
            for j in range(CH)
        ]
        for s in sps:
            s.wait()
        return 0

    lax.fori_loop(0, N_SUPER, body, 0)
    plsc.subcore_barrier()
    pltpu.sync_copy(acc_sh.at[sl], out.at[cid, sl])


@functools.lru_cache(maxsize=None)
def _scatter_call():
    return pl.kernel(
        _scatter_body,
        out_type=jax.ShapeDtypeStruct((2, N_PAD, _F), jnp.float32),
        mesh=_sc_mesh(),
        compiler_params=pltpu.CompilerParams(use_tc_tiling_on_sc=False),
        scratch_types=[
            pltpu.VMEM((CH, B), jnp.int32),
            pltpu.VMEM((CH, B), jnp.int32),
            pltpu.VMEM((CH, B, _F), jnp.float32),
            pltpu.VMEM_SHARED((N_PAD, _F), jnp.float32),
            pltpu.VMEM_SHARED((N_PAD, _F), jnp.float32),
            pltpu.SemaphoreType.DMA,
            pltpu.SemaphoreType.DMA,
        ],
    )


BN = 2048
G = N_PAD // BN


def _dinv_of(degs_ref):
    deg = degs_ref[0:1, :] + degs_ref[1:2, :] + 1.0
    return lax.rsqrt(deg)


def _matmul_t(wt, a):
    acc = wt[:, 0:1] * a[0:1, :]
    for k in range(1, wt.shape[1]):
        acc = acc + wt[:, k : k + 1] * a[k : k + 1, :]
    return acc


def _tc1_body(degs_ref, x_ref, w1t_ref, h1_ref, p1_ref):
    dinv = _dinv_of(degs_ref)
    h1 = _matmul_t(w1t_ref[...], x_ref[...])
    h1_ref[...] = h1
    p1_ref[...] = dinv * h1


def _tc2_body(acc_ref, degs_ref, h1_ref, w2t_ref, b1_ref, h2_ref, p2_ref):
    dinv = _dinv_of(degs_ref)
    accs = acc_ref[0] + acc_ref[1]
    z1 = dinv * accs + dinv * dinv * h1_ref[...] + b1_ref[...]
    a = jnp.maximum(z1, 0.0)
    h2 = _matmul_t(w2t_ref[...], a)
    h2_ref[...] = h2
    p2 = dinv * h2
    p2_ref[...] = jnp.concatenate([p2, jnp.zeros((6, BN), jnp.float32)], axis=0)


def _tc3_body(acc_ref, degs_ref, h2_ref, b2_ref, out_ref):
    dinv = _dinv_of(degs_ref)
    accs = acc_ref[0, :2, :] + acc_ref[1, :2, :]
    z = dinv * accs + dinv * dinv * h2_ref[...] + b2_ref[...]
    m = jnp.max(z, axis=0, keepdims=True)
    out_ref[...] = z - m - jnp.log(jnp.sum(jnp.exp(z - m), axis=0, keepdims=True))


def _col_spec(f):
    return pl.BlockSpec((f, BN), lambda i: (0, i))


def _acc_spec():
    return pl.BlockSpec((2, _F, BN), lambda i: (0, 0, i))


def _const_spec(shape):
    return pl.BlockSpec(shape, lambda i: tuple(0 for _ in shape))


def _tc1(degs, x_t, w1t):
    return pl.pallas_call(
        _tc1_body,
        grid=(G,),
        in_specs=[_col_spec(2), _col_spec(3), _const_spec((8, 3))],
        out_specs=[_col_spec(8), _col_spec(8)],
        out_shape=[
            jax.ShapeDtypeStruct((8, N_PAD), jnp.float32),
            jax.ShapeDtypeStruct((8, N_PAD), jnp.float32),
        ],
    )(degs, x_t, w1t)


def _tc2(acc, degs, h1, w2t, b1c):
    return pl.pallas_call(
        _tc2_body,
        grid=(G,),
        in_specs=[
            _acc_spec(), _col_spec(2), _col_spec(8),
            _const_spec((2, 8)), _const_spec((8, 1)),
        ],
        out_specs=[_col_spec(2), _col_spec(8)],
        out_shape=[
            jax.ShapeDtypeStruct((2, N_PAD), jnp.float32),
            jax.ShapeDtypeStruct((8, N_PAD), jnp.float32),
        ],
    )(acc, degs, h1, w2t, b1c)


def _tc3(acc, degs, h2, b2c):
    return pl.pallas_call(
        _tc3_body,
        grid=(G,),
        in_specs=[
            _acc_spec(), _col_spec(2), _col_spec(2), _const_spec((2, 1)),
        ],
        out_specs=_col_spec(2),
        out_shape=jax.ShapeDtypeStruct((2, N_PAD), jnp.float32),
    )(acc, degs, h2, b2c)


@jax.jit
def kernel(x, edge_index, W1, b1, W2, b2):
    ei = edge_index.astype(jnp.int32).reshape(2, EROWS, B)
    x_t = x.T
    zeros1 = jnp.zeros((ROWS_PER_TILE,), jnp.float32)
    zeros8 = jnp.zeros((ROWS_PER_TILE, _F), jnp.float32)
    ones = jnp.ones((B,), jnp.float32)

    degs = _deg_call()(ei, ones, zeros1)
    h1, p1 = _tc1(degs, x_t, W1.T)
    acc1 = _scatter_call()(ei, p1.T, zeros8)
    acc1_t = jnp.transpose(acc1, (0, 2, 1))
    h2, p2 = _tc2(acc1_t, degs, h1, W2.T, b1.reshape(8, 1))
    acc2 = _scatter_call()(ei, p2.T, zeros8)
    acc2_t = jnp.transpose(acc2, (0, 2, 1))
    out_t = _tc3(acc2_t, degs, h2, b2.reshape(2, 1))
    return out_t[:, :N].T


# speedup vs baseline: 152.8338x; 2.0711x over previous
import functools

import jax
import jax.numpy as jnp
from jax import lax
from jax.experimental import pallas as pl
from jax.experimental.pallas import tpu as pltpu
from jax.experimental.pallas import tpu_sc as plsc

N = 100000
E = 6400000
N_PAD = 100352
ROWS_PER_TILE = N_PAD // 16

NW = 32
B = 128
CH = 16
PER_W = E // NW
N_SUPER = PER_W // (CH * B)
REM = PER_W - N_SUPER * CH * B
REM_FULL = REM // B
TAIL = REM - REM_FULL * B

_F = 8


@functools.lru_cache(maxsize=None)
def _sc_mesh():
    return plsc.VectorSubcoreMesh(core_axis_name="c", subcore_axis_name="s")


def _deg_body(dst, ones, zeros, out, dst_v, ones_v, acc_sh, ssem):
    cid = lax.axis_index("c")
    sid = lax.axis_index("s")
    w = cid * 16 + sid
    sl = pl.ds(sid * ROWS_PER_TILE, ROWS_PER_TILE)
    pltpu.sync_copy(zeros, acc_sh.at[sl])
    pltpu.sync_copy(ones, ones_v)
    plsc.subcore_barrier()
    base = w * PER_W

    def scatter_block(ndma, tail):
        def go(off):
            n = ndma * B + tail
            pltpu.sync_copy(dst.at[pl.ds(off, n)], dst_v.at[pl.ds(0, n)])
            sps = [
                pltpu.async_copy(
                    ones_v, acc_sh.at[dst_v.at[pl.ds(j * B, B)]], ssem, add=True
                )
                for j in range(ndma)
            ]
            if tail:
                sps.append(
                    pltpu.async_copy(
                        ones_v.at[pl.ds(0, tail)],
                        acc_sh.at[dst_v.at[pl.ds(ndma * B, tail)]],
                        ssem, add=True,
                    )
                )
            for s in sps:
                s.wait()
        return go

    full = scatter_block(CH, 0)

    def body(i, _):
        full(base + i * (CH * B))
        return 0

    lax.fori_loop(0, N_SUPER, body, 0)
    scatter_block(REM_FULL, TAIL)(base + N_SUPER * CH * B)
    plsc.subcore_barrier()
    pltpu.sync_copy(acc_sh.at[sl], out.at[cid, sl])


@functools.lru_cache(maxsize=None)
def _deg_call():
    return pl.kernel(
        _deg_body,
        out_type=jax.ShapeDtypeStruct((2, N_PAD), jnp.float32),
        mesh=_sc_mesh(),
        compiler_params=pltpu.CompilerParams(use_tc_tiling_on_sc=False),
        scratch_types=[
            pltpu.VMEM((CH * B,), jnp.int32),
            pltpu.VMEM((B,), jnp.float32),
            pltpu.VMEM_SHARED((N_PAD,), jnp.float32),
            pltpu.SemaphoreType.DMA,
        ],
    )


def _scatter_body(
    src, dst, p, zeros, out,
    src_v, dst_v, rows_v, tab_sh, acc_sh, gsem, ssem,
):
    cid = lax.axis_index("c")
    sid = lax.axis_index("s")
    w = cid * 16 + sid
    sl = pl.ds(sid * ROWS_PER_TILE, ROWS_PER_TILE)
    pltpu.sync_copy(zeros, acc_sh.at[sl])
    pltpu.sync_copy(p.at[sl], tab_sh.at[sl])
    plsc.subcore_barrier()
    base = w * PER_W

    def block(ndma, tail):
        def go(off):
            n = ndma * B + tail
            i0 = pltpu.async_copy(src.at[pl.ds(off, n)], src_v.at[pl.ds(0, n)], gsem)
            i1 = pltpu.async_copy(dst.at[pl.ds(off, n)], dst_v.at[pl.ds(0, n)], gsem)
            i0.wait()
            i1.wait()
            sizes = [B] * ndma + ([tail] if tail else [])
            cps = [
                pltpu.async_copy(
                    tab_sh.at[src_v.at[pl.ds(j * B, s)]],
                    rows_v.at[pl.ds(j * B, s)], gsem,
                )
                for j, s in enumerate(sizes)
            ]
            for c in cps:
                c.wait()
            sps = [
                pltpu.async_copy(
                    rows_v.at[pl.ds(j * B, s)],
                    acc_sh.at[dst_v.at[pl.ds(j * B, s)]], ssem, add=True,
                )
                for j, s in enumerate(sizes)
            ]
            for s_ in sps:
                s_.wait()
        return go

    full = block(CH, 0)

    def body(i, _):
        full(base + i * (CH * B))
        return 0

    lax.fori_loop(0, N_SUPER, body, 0)
    block(REM_FULL, TAIL)(base + N_SUPER * CH * B)
    plsc.subcore_barrier()
    pltpu.sync_copy(acc_sh.at[sl], out.at[cid, sl])


@functools.lru_cache(maxsize=None)
def _scatter_call():
    return pl.kernel(
        _scatter_body,
        out_type=jax.ShapeDtypeStruct((2, N_PAD, _F), jnp.float32),
        mesh=_sc_mesh(),
        compiler_params=pltpu.CompilerParams(use_tc_tiling_on_sc=False),
        scratch_types=[
            pltpu.VMEM((CH * B,), jnp.int32),
            pltpu.VMEM((CH * B,), jnp.int32),
            pltpu.VMEM((CH * B, _F), jnp.float32),
            pltpu.VMEM_SHARED((N_PAD, _F), jnp.float32),
            pltpu.VMEM_SHARED((N_PAD, _F), jnp.float32),
            pltpu.SemaphoreType.DMA,
            pltpu.SemaphoreType.DMA,
        ],
    )


BN = 2048
G = N_PAD // BN


def _dinv_of(degs_ref):
    deg = degs_ref[0:1, :] + degs_ref[1:2, :] + 1.0
    return lax.rsqrt(deg)


def _matmul_t(wt, a):
    acc = wt[:, 0:1] * a[0:1, :]
    for k in range(1, wt.shape[1]):
        acc = acc + wt[:, k : k + 1] * a[k : k + 1, :]
    return acc


def _tc1_body(degs_ref, x_ref, w1t_ref, h1_ref, p1_ref):
    dinv = _dinv_of(degs_ref)
    h1 = _matmul_t(w1t_ref[...], x_ref[...])
    h1_ref[...] = h1
    p1_ref[...] = dinv * h1


def _tc2_body(acc_ref, degs_ref, h1_ref, w2t_ref, b1_ref, h2_ref, p2_ref):
    dinv = _dinv_of(degs_ref)
    accs = acc_ref[0] + acc_ref[1]
    z1 = dinv * accs + dinv * dinv * h1_ref[...] + b1_ref[...]
    a = jnp.maximum(z1, 0.0)
    h2 = _matmul_t(w2t_ref[...], a)
    h2_ref[...] = h2
    p2 = dinv * h2
    p2_ref[...] = jnp.concatenate([p2, jnp.zeros((6, BN), jnp.float32)], axis=0)


def _tc3_body(acc_ref, degs_ref, h2_ref, b2_ref, out_ref):
    dinv = _dinv_of(degs_ref)
    accs = acc_ref[0, :2, :] + acc_ref[1, :2, :]
    z = dinv * accs + dinv * dinv * h2_ref[...] + b2_ref[...]
    m = jnp.max(z, axis=0, keepdims=True)
    out_ref[...] = z - m - jnp.log(jnp.sum(jnp.exp(z - m), axis=0, keepdims=True))


def _col_spec(f):
    return pl.BlockSpec((f, BN), lambda i: (0, i))


def _acc_spec():
    return pl.BlockSpec((2, _F, BN), lambda i: (0, 0, i))


def _const_spec(shape):
    return pl.BlockSpec(shape, lambda i: tuple(0 for _ in shape))


def _tc1(degs, x_t, w1t):
    return pl.pallas_call(
        _tc1_body,
        grid=(G,),
        in_specs=[_col_spec(2), _col_spec(3), _const_spec((8, 3))],
        out_specs=[_col_spec(8), _col_spec(8)],
        out_shape=[
            jax.ShapeDtypeStruct((8, N_PAD), jnp.float32),
            jax.ShapeDtypeStruct((8, N_PAD), jnp.float32),
        ],
    )(degs, x_t, w1t)


def _tc2(acc, degs, h1, w2t, b1c):
    return pl.pallas_call(
        _tc2_body,
        grid=(G,),
        in_specs=[
            _acc_spec(), _col_spec(2), _col_spec(8),
            _const_spec((2, 8)), _const_spec((8, 1)),
        ],
        out_specs=[_col_spec(2), _col_spec(8)],
        out_shape=[
            jax.ShapeDtypeStruct((2, N_PAD), jnp.float32),
            jax.ShapeDtypeStruct((8, N_PAD), jnp.float32),
        ],
    )(acc, degs, h1, w2t, b1c)


def _tc3(acc, degs, h2, b2c):
    return pl.pallas_call(
        _tc3_body,
        grid=(G,),
        in_specs=[
            _acc_spec(), _col_spec(2), _col_spec(2), _const_spec((2, 1)),
        ],
        out_specs=_col_spec(2),
        out_shape=jax.ShapeDtypeStruct((2, N_PAD), jnp.float32),
    )(acc, degs, h2, b2c)


@jax.jit
def kernel(x, edge_index, W1, b1, W2, b2):
    ei = edge_index.astype(jnp.int32)
    src = ei[0]
    dst = ei[1]
    x_t = x.T
    zeros1 = jnp.zeros((ROWS_PER_TILE,), jnp.float32)
    zeros8 = jnp.zeros((ROWS_PER_TILE, _F), jnp.float32)
    ones = jnp.ones((B,), jnp.float32)

    degs = _deg_call()(dst, ones, zeros1)
    h1, p1 = _tc1(degs, x_t, W1.T)
    acc1 = _scatter_call()(src, dst, p1.T, zeros8)
    acc1_t = jnp.transpose(acc1, (0, 2, 1))
    h2, p2 = _tc2(acc1_t, degs, h1, W2.T, b1.reshape(8, 1))
    acc2 = _scatter_call()(src, dst, p2.T, zeros8)
    acc2_t = jnp.transpose(acc2, (0, 2, 1))
    out_t = _tc3(acc2_t, degs, h2, b2.reshape(2, 1))
    return out_t[:, :N].T
